# baseline (speedup 1.0000x reference)
import jax
import jax.numpy as jnp
from jax import lax
from jax.experimental import pallas as pl
from jax.experimental.pallas import tpu as pltpu
from jax.experimental.pallas import tpu_sc as plsc

NC = 2
NS = 16
L = 16
NW = NC * NS

BATCH = 16384
DIM = 32
G = 4
S = 8
TW = 128
RS = 4
NSLOT = 3
EXTS = (32, 64, 128, 128)
BPW = BATCH // NW
NSET = BPW // RS
PPW = 2 * BPW


def _body(u_hbm, i_hbm, wtu_hbm, wti_hbm, out_hbm,
          uidx_v, iidx_v, ublk, iblk, opad_v, out_v, sem0):
    wid = lax.axis_index("s") * NC + lax.axis_index("c")

    pltpu.sync_copy(u_hbm.at[pl.ds(wid * PPW, PPW)], uidx_v.at[pl.ds(0, PPW)])
    pltpu.sync_copy(i_hbm.at[pl.ds(wid * PPW, PPW)], iidx_v.at[pl.ds(0, PPW)])

    lane = lax.iota(jnp.int32, L)
    set_mask = lane < RS

    def fire(k, slot):
        off = pl.multiple_of(k * 2 * RS, 8)
        ruv = uidx_v[pl.ds(off, L)]
        riv = iidx_v[pl.ds(off, L)]
        for j in range(RS):
            for src, idx, blk in ((wtu_hbm, ruv, ublk), (wti_hbm, riv, iblk)):
                r = idx[j]
                rt = pl.multiple_of(r - (r & (TW - 1)), TW)
                band = lax.shift_right_logical(r & (TW - 1), 5)
                for bb in range(len(EXTS)):

                    @pl.when(band == bb)
                    def _copy(src=src, blk=blk, rt=rt, j=j, ext=EXTS[bb]):
                        pltpu.async_copy(
                            src.at[:, :, pl.ds(rt, ext)],
                            blk.at[slot, j, :, :, pl.ds(0, ext)], sem0)

    def step(k, _):
        @pl.when(k + 2 < NSET)
        def _fire_next():
            nxt = k + 2
            fire(nxt, nxt - (nxt // NSLOT) * NSLOT)

        doff = pl.multiple_of(k * 2 * RS, 8)
        druv = uidx_v[pl.ds(doff, L)]
        driv = iidx_v[pl.ds(doff, L)]
        for j in range(RS):
            for idx in (druv, driv):
                band = lax.shift_right_logical(idx[j] & (TW - 1), 5)
                for bb in range(len(EXTS)):

                    @pl.when(band == bb)
                    def _wait(ext=EXTS[bb]):
                        pltpu.make_async_copy(
                            wtu_hbm.at[:, :, pl.ds(0, ext)],
                            ublk.at[0, 0, :, :, pl.ds(0, ext)], sem0).wait()

        off = pl.multiple_of(k * 2 * RS, 8)
        ruv = uidx_v[pl.ds(off, L)]
        riv = iidx_v[pl.ds(off, L)]
        slot = k - (k // NSLOT) * NSLOT
        slotv = jnp.zeros((L,), jnp.int32) + slot
        rowv = lane & (RS - 1)
        colu = ruv & (TW - 1)
        coli = riv & (TW - 1)
        acc = jnp.zeros((L,), jnp.float32)
        for gg in range(G):
            ggv = jnp.full((L,), gg, jnp.int32)
            for s in range(S):
                sv = jnp.full((L,), s, jnp.int32)
                acc = acc + (plsc.load_gather(ublk, [slotv, rowv, ggv, sv, colu])
                             * plsc.load_gather(iblk, [slotv, rowv, ggv, sv, coli]))
        plsc.store_compressed(opad_v.at[pl.ds(off, L)], acc, mask=set_mask)
        return _

    fire(0, 0)
    fire(1, 1)
    lax.fori_loop(0, NSET, step, 0)

    def compact(g, _):
        src = g * 2 * L + lax.shift_right_logical(lane, 2) * 2 * RS + (lane & (RS - 1))
        out_v[pl.ds(pl.multiple_of(g * L, L), L)] = plsc.load_gather(opad_v, [src])
        return _

    lax.fori_loop(0, BPW // L, compact, 0)

    pltpu.sync_copy(out_v, out_hbm.at[pl.ds(wid * BPW, BPW)])


def kernel(u, i, user_weight, item_weight):
    u32 = u.astype(jnp.int32)
    i32 = i.astype(jnp.int32)
    up = jnp.pad(u32.reshape(-1, RS), ((0, 0), (0, 8 - RS))).reshape(-1)
    ip = jnp.pad(i32.reshape(-1, RS), ((0, 0), (0, 8 - RS))).reshape(-1)
    wtu = user_weight.T.reshape(G, S, -1)
    wti = item_weight.T.reshape(G, S, -1)
    mesh = plsc.VectorSubcoreMesh(core_axis_name="c", subcore_axis_name="s",
                                  num_cores=NC, num_subcores=NS)
    f = pl.kernel(
        _body,
        out_type=jax.ShapeDtypeStruct((BATCH,), jnp.float32),
        mesh=mesh,
        compiler_params=pltpu.CompilerParams(needs_layout_passes=False,
                                             use_tc_tiling_on_sc=True),
        scratch_types=[
            pltpu.VMEM((PPW + 8,), jnp.int32),
            pltpu.VMEM((PPW + 8,), jnp.int32),
            pltpu.VMEM((NSLOT, RS, G, S, TW), jnp.float32),
            pltpu.VMEM((NSLOT, RS, G, S, TW), jnp.float32),
            pltpu.VMEM((PPW + 8,), jnp.float32),
            pltpu.VMEM((BPW,), jnp.float32),
            pltpu.SemaphoreType.DMA,
        ],
    )
    return f(up, ip, wtu, wti)


# speedup vs baseline: 1.0221x; 1.0221x over previous
import jax
import jax.numpy as jnp
from jax import lax
from jax.experimental import pallas as pl
from jax.experimental.pallas import tpu as pltpu
from jax.experimental.pallas import tpu_sc as plsc

NC = 2
NS = 16
L = 16
NW = NC * NS

BATCH = 16384
DIM = 32
G = 4
S = 8
TW = 128
RS = 4
NSLOT = 3
EXTS = (64, 128)
BPW = BATCH // NW
NSET = BPW // RS
PPW = 2 * BPW


def _body(u_hbm, i_hbm, wtu_hbm, wti_hbm, out_hbm,
          uidx_v, iidx_v, ublk, iblk, opad_v, out_v, sem0):
    wid = lax.axis_index("s") * NC + lax.axis_index("c")

    pltpu.sync_copy(u_hbm.at[pl.ds(wid * PPW, PPW)], uidx_v.at[pl.ds(0, PPW)])
    pltpu.sync_copy(i_hbm.at[pl.ds(wid * PPW, PPW)], iidx_v.at[pl.ds(0, PPW)])

    lane = lax.iota(jnp.int32, L)
    set_mask = lane < RS

    def fire(k, slot):
        off = pl.multiple_of(k * 2 * RS, 8)
        ruv = uidx_v[pl.ds(off, L)]
        riv = iidx_v[pl.ds(off, L)]
        for j in range(RS):
            for src, idx, blk in ((wtu_hbm, ruv, ublk), (wti_hbm, riv, iblk)):
                r = idx[j]
                rt = pl.multiple_of(r - (r & (TW - 1)), TW)
                band = lax.shift_right_logical(r & (TW - 1), 6)
                for bb in range(len(EXTS)):

                    @pl.when(band == bb)
                    def _copy(src=src, blk=blk, rt=rt, j=j, ext=EXTS[bb]):
                        pltpu.async_copy(
                            src.at[:, :, pl.ds(rt, ext)],
                            blk.at[slot, j, :, :, pl.ds(0, ext)], sem0)

    def step(k, _):
        @pl.when(k + 2 < NSET)
        def _fire_next():
            nxt = k + 2
            fire(nxt, nxt - (nxt // NSLOT) * NSLOT)

        doff = pl.multiple_of(k * 2 * RS, 8)
        druv = uidx_v[pl.ds(doff, L)]
        driv = iidx_v[pl.ds(doff, L)]
        for j in range(RS):
            for idx in (druv, driv):
                band = lax.shift_right_logical(idx[j] & (TW - 1), 6)
                for bb in range(len(EXTS)):

                    @pl.when(band == bb)
                    def _wait(ext=EXTS[bb]):
                        pltpu.make_async_copy(
                            wtu_hbm.at[:, :, pl.ds(0, ext)],
                            ublk.at[0, 0, :, :, pl.ds(0, ext)], sem0).wait()

        off = pl.multiple_of(k * 2 * RS, 8)
        ruv = uidx_v[pl.ds(off, L)]
        riv = iidx_v[pl.ds(off, L)]
        slot = k - (k // NSLOT) * NSLOT
        slotv = jnp.zeros((L,), jnp.int32) + slot
        rowv = lane & (RS - 1)
        colu = ruv & (TW - 1)
        coli = riv & (TW - 1)
        acc = jnp.zeros((L,), jnp.float32)
        for gg in range(G):
            ggv = jnp.full((L,), gg, jnp.int32)
            for s in range(S):
                sv = jnp.full((L,), s, jnp.int32)
                acc = acc + (plsc.load_gather(ublk, [slotv, rowv, ggv, sv, colu])
                             * plsc.load_gather(iblk, [slotv, rowv, ggv, sv, coli]))
        plsc.store_compressed(opad_v.at[pl.ds(off, L)], acc, mask=set_mask)
        return _

    fire(0, 0)
    fire(1, 1)
    lax.fori_loop(0, NSET, step, 0)

    def compact(g, _):
        src = g * 2 * L + lax.shift_right_logical(lane, 2) * 2 * RS + (lane & (RS - 1))
        out_v[pl.ds(pl.multiple_of(g * L, L), L)] = plsc.load_gather(opad_v, [src])
        return _

    lax.fori_loop(0, BPW // L, compact, 0)

    pltpu.sync_copy(out_v, out_hbm.at[pl.ds(wid * BPW, BPW)])


def kernel(u, i, user_weight, item_weight):
    u32 = u.astype(jnp.int32)
    i32 = i.astype(jnp.int32)
    up = jnp.pad(u32.reshape(-1, RS), ((0, 0), (0, 8 - RS))).reshape(-1)
    ip = jnp.pad(i32.reshape(-1, RS), ((0, 0), (0, 8 - RS))).reshape(-1)
    wtu = user_weight.T.reshape(G, S, -1)
    wti = item_weight.T.reshape(G, S, -1)
    mesh = plsc.VectorSubcoreMesh(core_axis_name="c", subcore_axis_name="s",
                                  num_cores=NC, num_subcores=NS)
    f = pl.kernel(
        _body,
        out_type=jax.ShapeDtypeStruct((BATCH,), jnp.float32),
        mesh=mesh,
        compiler_params=pltpu.CompilerParams(needs_layout_passes=False,
                                             use_tc_tiling_on_sc=True),
        scratch_types=[
            pltpu.VMEM((PPW + 8,), jnp.int32),
            pltpu.VMEM((PPW + 8,), jnp.int32),
            pltpu.VMEM((NSLOT, RS, G, S, TW), jnp.float32),
            pltpu.VMEM((NSLOT, RS, G, S, TW), jnp.float32),
            pltpu.VMEM((PPW + 8,), jnp.float32),
            pltpu.VMEM((BPW,), jnp.float32),
            pltpu.SemaphoreType.DMA,
        ],
    )
    return f(up, ip, wtu, wti)
